# baseline (speedup 1.0000x reference)
import functools

import jax
import jax.numpy as jnp
from jax import lax
from jax.experimental import pallas as pl
from jax.experimental.pallas import tpu as pltpu
from jax.experimental.pallas import tpu_sc as plsc

NC = 2
NS = 16
NW = NC * NS
LANES = 16
B = 16384
D = 32
BPW = B // NW
CH = 128
NCH = BPW // CH


def _mfmf_body(uid_hbm, iid_hbm, ue_hbm, ve_hbm, ie_hbm, zm_hbm, out_hbm,
               uid_v, iid_v, u_v, v_v, i_v, z_v, out_v, sem):
    wid = lax.axis_index("s") * NC + lax.axis_index("c")
    base = wid * BPW

    pltpu.sync_copy(uid_hbm.at[wid], uid_v)
    pltpu.sync_copy(iid_hbm.at[wid], iid_v)

    copies = []
    for c in range(NCH):
        dst = pl.ds(c * CH, CH)
        copies.append(pltpu.async_copy(ue_hbm.at[uid_v.at[c]], u_v.at[dst], sem))
        copies.append(pltpu.async_copy(ve_hbm.at[iid_v.at[c]], v_v.at[dst], sem))
        copies.append(pltpu.async_copy(ie_hbm.at[iid_v.at[c]], i_v.at[dst], sem))
        copies.append(pltpu.async_copy(zm_hbm.at[uid_v.at[c]], z_v.at[dst], sem))
    for cp in copies:
        cp.wait()

    lane = lax.iota(jnp.int32, LANES)
    perms = [lane ^ (1 << k) for k in range(4)]
    _dnums = lax.GatherDimensionNumbers(
        offset_dims=(), collapsed_slice_dims=(0,), start_index_map=(0,))

    def _permute(x, idx):
        return lax.gather(
            x, idx[:, None], _dnums, (1,),
            mode=lax.GatherScatterMode.PROMISE_IN_BOUNDS)
    lo = pl.ds(0, LANES)
    hi = pl.ds(LANES, LANES)

    def group(g, carry):
        acc = jnp.zeros((LANES,), jnp.float32)
        for l in range(LANES):
            b = g * LANES + l
            t = (u_v[b, lo] * v_v[b, lo] + u_v[b, hi] * v_v[b, hi]
                 + i_v[b, lo] * z_v[b, lo] + i_v[b, hi] * z_v[b, hi])
            for p in perms:
                t = t + _permute(t, p)
            acc = jnp.where(lane == l, t, acc)
        out_v[pl.ds(g * LANES, LANES)] = acc
        return carry

    lax.fori_loop(0, BPW // LANES, group, 0)

    pltpu.sync_copy(out_v, out_hbm.at[pl.ds(base, BPW)])


_mfmf = functools.partial(
    pl.kernel,
    mesh=plsc.VectorSubcoreMesh(core_axis_name="c", subcore_axis_name="s"),
    compiler_params=pltpu.CompilerParams(use_tc_tiling_on_sc=False),
    out_type=jax.ShapeDtypeStruct((B,), jnp.float32),
    scratch_types=[
        pltpu.VMEM((NCH, CH), jnp.int32),
        pltpu.VMEM((NCH, CH), jnp.int32),
        pltpu.VMEM((BPW, D), jnp.float32),
        pltpu.VMEM((BPW, D), jnp.float32),
        pltpu.VMEM((BPW, D), jnp.float32),
        pltpu.VMEM((BPW, D), jnp.float32),
        pltpu.VMEM((BPW,), jnp.float32),
        pltpu.SemaphoreType.DMA,
    ],
)(_mfmf_body)


def kernel(uid, iid, mf_user_emb, mf_item_emb, item_emb, ivae_mean):
    uid3 = uid.reshape(NW, NCH, CH)
    iid3 = iid.reshape(NW, NCH, CH)
    return _mfmf(uid3, iid3, mf_user_emb, mf_item_emb, item_emb, ivae_mean)
